# baseline (speedup 1.0000x reference)
import functools

import jax
import jax.numpy as jnp
from jax import lax
from jax.experimental import pallas as pl
from jax.experimental.pallas import tpu as pltpu
from jax.experimental.pallas import tpu_sc as plsc

N_NODES = 10242
N_EDGES = 81900
D = 512
DC = 128
NCH = D // DC
E_PAD = 81920
N_PAD = 10752
EB = 1024
NB = 512
NLAYER = 9


def _silu(x):
    return x * (1.0 / (1.0 + jnp.exp(-x)))


def _ln(h, s, b):
    mu = jnp.mean(h, axis=-1, keepdims=True)
    d = h - mu
    var = jnp.mean(d * d, axis=-1, keepdims=True)
    return d * lax.rsqrt(var + 1e-5) * s + b


def _proj_body(nodes_ref, w1b_ref, w1c_ref, p_ref, q_ref):
    x = nodes_ref[...]
    p_ref[...] = jnp.dot(x, w1b_ref[...],
                         preferred_element_type=jnp.float32).astype(jnp.bfloat16)
    q_ref[...] = jnp.dot(x, w1c_ref[...],
                         preferred_element_type=jnp.float32).astype(jnp.bfloat16)


def _tc_proj(nodes, w1b, w1c):
    grid = (N_PAD // NB,)
    return pl.pallas_call(
        _proj_body,
        grid=grid,
        in_specs=[
            pl.BlockSpec((NB, D), lambda i: (i, 0)),
            pl.BlockSpec((D, D), lambda i: (0, 0)),
            pl.BlockSpec((D, D), lambda i: (0, 0)),
        ],
        out_specs=[
            pl.BlockSpec((NB, D), lambda i: (i, 0)),
            pl.BlockSpec((NB, D), lambda i: (i, 0)),
        ],
        out_shape=[
            jax.ShapeDtypeStruct((N_PAD, D), jnp.bfloat16),
            jax.ShapeDtypeStruct((N_PAD, D), jnp.bfloat16),
        ],
    )(nodes, w1b, w1c)


def _edge_body(e_ref, pg_ref, qg_ref, w1a_ref, b1_ref, w2_ref, b2_ref,
               s_ref, b_ref, out_ref):
    i = pl.program_id(0)
    e = jnp.concatenate([e_ref[c] for c in range(NCH)], axis=-1)
    x = jnp.dot(e, w1a_ref[...], preferred_element_type=jnp.float32)
    x = (x + pg_ref[...].astype(jnp.float32) + qg_ref[...].astype(jnp.float32)
         + b1_ref[...])
    h = _silu(x)
    y = jnp.dot(h, w2_ref[...], preferred_element_type=jnp.float32) + b2_ref[...]
    new = e + _ln(y, s_ref[...], b_ref[...])
    rows = i * EB + lax.broadcasted_iota(jnp.int32, (EB, 1), 0)
    new = jnp.where(rows < N_EDGES, new, 0.0)
    for c in range(NCH):
        out_ref[c] = new[:, c * DC:(c + 1) * DC]


def _tc_edge(edges_cm, pg, qg, w1a, b1, w2, b2, s, b):
    grid = (E_PAD // EB,)
    return pl.pallas_call(
        _edge_body,
        grid=grid,
        in_specs=[
            pl.BlockSpec((NCH, EB, DC), lambda i: (0, i, 0)),
            pl.BlockSpec((EB, D), lambda i: (i, 0)),
            pl.BlockSpec((EB, D), lambda i: (i, 0)),
            pl.BlockSpec((D, D), lambda i: (0, 0)),
            pl.BlockSpec((1, D), lambda i: (0, 0)),
            pl.BlockSpec((D, D), lambda i: (0, 0)),
            pl.BlockSpec((1, D), lambda i: (0, 0)),
            pl.BlockSpec((1, D), lambda i: (0, 0)),
            pl.BlockSpec((1, D), lambda i: (0, 0)),
        ],
        out_specs=pl.BlockSpec((NCH, EB, DC), lambda i: (0, i, 0)),
        out_shape=jax.ShapeDtypeStruct((NCH, E_PAD, DC), jnp.float32),
    )(edges_cm, pg, qg, w1a, b1, w2, b2, s, b)


def _node_body(n_ref, agg_ref, wn1a_ref, wn1b_ref, b1_ref, wn2_ref, b2_ref,
               s_ref, b_ref, out_ref):
    n = n_ref[...]
    agg = jnp.concatenate([agg_ref[c] for c in range(NCH)], axis=-1)
    x = (jnp.dot(n, wn1a_ref[...], preferred_element_type=jnp.float32)
         + jnp.dot(agg, wn1b_ref[...], preferred_element_type=jnp.float32)
         + b1_ref[...])
    h = _silu(x)
    y = jnp.dot(h, wn2_ref[...], preferred_element_type=jnp.float32) + b2_ref[...]
    out_ref[...] = n + _ln(y, s_ref[...], b_ref[...])


def _tc_node(nodes, agg_cm, wn1a, wn1b, b1, wn2, b2, s, b):
    grid = (N_PAD // NB,)
    return pl.pallas_call(
        _node_body,
        grid=grid,
        in_specs=[
            pl.BlockSpec((NB, D), lambda i: (i, 0)),
            pl.BlockSpec((NCH, NB, DC), lambda i: (0, i, 0)),
            pl.BlockSpec((D, D), lambda i: (0, 0)),
            pl.BlockSpec((D, D), lambda i: (0, 0)),
            pl.BlockSpec((1, D), lambda i: (0, 0)),
            pl.BlockSpec((D, D), lambda i: (0, 0)),
            pl.BlockSpec((1, D), lambda i: (0, 0)),
            pl.BlockSpec((1, D), lambda i: (0, 0)),
            pl.BlockSpec((1, D), lambda i: (0, 0)),
        ],
        out_specs=pl.BlockSpec((NB, D), lambda i: (i, 0)),
        out_shape=jax.ShapeDtypeStruct((N_PAD, D), jnp.float32),
    )(nodes, agg_cm, wn1a, wn1b, b1, wn2, b2, s, b)


_NW = 32
_GK = 40
_SK = 128
_SLAB = N_PAD // 16
_SB = 96


def _gather_body(p_hbm, q_hbm, src_hbm, dst_hbm, pg_hbm, qg_hbm,
                 sidx_v, didx_v, pb0, pb1, qb0, qb1,
                 sp0, sp1, sq0, sq1):
    per_w = E_PAD // _NW
    n_iter = per_w // _GK // 2
    wid = lax.axis_index("s") * 2 + lax.axis_index("c")
    base0 = wid * per_w

    pltpu.sync_copy(src_hbm.at[pl.ds(base0, per_w)], sidx_v)
    pltpu.sync_copy(dst_hbm.at[pl.ds(base0, per_w)], didx_v)

    pbufs, qbufs = (pb0, pb1), (qb0, qb1)
    psems, qsems = (sp0, sp1), (sq0, sq1)

    def body(i, carry):
        for b in range(2):
            c = 2 * i + b
            off = c * _GK
            @pl.when(i > 0)
            def _():
                pltpu.make_async_copy(
                    pbufs[b], pg_hbm.at[pl.ds(0, _GK)], psems[b]).wait()
                pltpu.make_async_copy(
                    qbufs[b], qg_hbm.at[pl.ds(0, _GK)], qsems[b]).wait()
            pltpu.async_copy(p_hbm.at[sidx_v.at[pl.ds(off, _GK)]],
                             pbufs[b], psems[b])
            pltpu.async_copy(q_hbm.at[didx_v.at[pl.ds(off, _GK)]],
                             qbufs[b], qsems[b])
        for b in range(2):
            c = 2 * i + b
            base = base0 + c * _GK
            pltpu.make_async_copy(p_hbm.at[pl.ds(0, _GK)], pbufs[b],
                                  psems[b]).wait()
            pltpu.async_copy(pbufs[b], pg_hbm.at[pl.ds(base, _GK)], psems[b])
            pltpu.make_async_copy(q_hbm.at[pl.ds(0, _GK)], qbufs[b],
                                  qsems[b]).wait()
            pltpu.async_copy(qbufs[b], qg_hbm.at[pl.ds(base, _GK)], qsems[b])
        return carry

    lax.fori_loop(0, n_iter, body, 0)
    for b in range(2):
        pltpu.make_async_copy(pbufs[b], pg_hbm.at[pl.ds(0, _GK)],
                              psems[b]).wait()
        pltpu.make_async_copy(qbufs[b], qg_hbm.at[pl.ds(0, _GK)],
                              qsems[b]).wait()


def _sc_gather(p, q, src, dst):
    mesh = plsc.VectorSubcoreMesh(core_axis_name="c", subcore_axis_name="s")
    per_w = E_PAD // _NW
    fn = pl.kernel(
        _gather_body,
        out_type=[
            jax.ShapeDtypeStruct((E_PAD, D // 2), jnp.int32),
            jax.ShapeDtypeStruct((E_PAD, D // 2), jnp.int32),
        ],
        mesh=mesh,
        scratch_types=[
            pltpu.VMEM((per_w,), jnp.int32),
            pltpu.VMEM((per_w,), jnp.int32),
            pltpu.VMEM((_GK, D // 2), jnp.int32),
            pltpu.VMEM((_GK, D // 2), jnp.int32),
            pltpu.VMEM((_GK, D // 2), jnp.int32),
            pltpu.VMEM((_GK, D // 2), jnp.int32),
            pltpu.SemaphoreType.DMA,
            pltpu.SemaphoreType.DMA,
            pltpu.SemaphoreType.DMA,
            pltpu.SemaphoreType.DMA,
        ],
    )
    return fn(p, q, src, dst)


def _scatter_body(e_hbm, dst2d_hbm, agg_hbm, r0_v, r1_v, idx_v, spmem,
                  sa0, sa1, sd0, sd1):
    cid = lax.axis_index("c")
    sid = lax.axis_index("s")
    per_tile = E_PAD // 16
    n_iter = per_tile // _SK
    n_sl = _SLAB // _SB

    rows = (r0_v, r1_v)
    asems = (sa0, sa1)
    dsems = (sd0, sd1)

    z16 = jnp.zeros((16,), jnp.float32)

    def zrow(i, carry):
        for j in range(DC // 16):
            r0_v[i, pl.ds(j * 16, 16)] = z16
        return carry

    pltpu.sync_copy(dst2d_hbm.at[pl.ds(sid * n_iter, n_iter)], idx_v)

    for j in range(2):
        chunk = cid * 2 + j

        lax.fori_loop(0, _SK, zrow, 0)
        base_r = sid * _SLAB
        for t in range(_SLAB // _SK):
            pltpu.sync_copy(r0_v, spmem.at[pl.ds(base_r + t * _SK, _SK), :])
        rem = _SLAB % _SK
        if rem:
            pltpu.sync_copy(r0_v.at[pl.ds(0, rem)],
                            spmem.at[pl.ds(base_r + _SLAB - rem, rem), :])
        plsc.subcore_barrier()

        def body(i, carry):
            for b in range(2):
                c = 2 * i + b
                base = sid * per_tile + c * _SK

                @pl.when(i > 0)
                def _():
                    pltpu.make_async_copy(
                        rows[b], spmem.at[pl.ds(0, _SK)], asems[b]).wait()

                pltpu.sync_copy(e_hbm.at[chunk, pl.ds(base, _SK)], rows[b])
                pltpu.async_copy(rows[b], spmem.at[idx_v.at[c]], asems[b],
                                 add=True)
            return carry

        lax.fori_loop(0, n_iter // 2, body, 0)
        for b in range(2):
            pltpu.make_async_copy(rows[b], spmem.at[pl.ds(0, _SK)],
                                  asems[b]).wait()
        plsc.subcore_barrier()

        def drain(t, carry):
            for b in range(2):
                tt = 2 * t + b
                r0 = sid * _SLAB + tt * _SB

                @pl.when(tt < n_sl)
                def _():
                    @pl.when(t > 0)
                    def _():
                        pltpu.make_async_copy(
                            rows[b].at[pl.ds(0, _SB)],
                            agg_hbm.at[chunk, pl.ds(0, _SB)], dsems[b]).wait()
                    pltpu.sync_copy(spmem.at[pl.ds(r0, _SB), :],
                                    rows[b].at[pl.ds(0, _SB)])
                    pltpu.async_copy(rows[b].at[pl.ds(0, _SB)],
                                     agg_hbm.at[chunk, pl.ds(r0, _SB)],
                                     dsems[b])
            return carry

        lax.fori_loop(0, (n_sl + 1) // 2, drain, 0)
        for b in range(2):
            pltpu.make_async_copy(rows[b].at[pl.ds(0, _SB)],
                                  agg_hbm.at[chunk, pl.ds(0, _SB)],
                                  dsems[b]).wait()
        plsc.subcore_barrier()


def _sc_scatter(edges_cm, dst2d):
    mesh = plsc.VectorSubcoreMesh(core_axis_name="c", subcore_axis_name="s")
    fn = pl.kernel(
        _scatter_body,
        out_type=jax.ShapeDtypeStruct((NCH, N_PAD, DC), jnp.float32),
        mesh=mesh,
        scratch_types=[
            pltpu.VMEM((_SK, DC), jnp.float32),
            pltpu.VMEM((_SK, DC), jnp.float32),
            pltpu.VMEM((E_PAD // 16 // _SK, _SK), jnp.int32),
            pltpu.VMEM_SHARED((N_PAD, DC), jnp.float32),
            pltpu.SemaphoreType.DMA,
            pltpu.SemaphoreType.DMA,
            pltpu.SemaphoreType.DMA,
            pltpu.SemaphoreType.DMA,
        ],
    )
    return fn(edges_cm, dst2d)


def kernel(embedded_mesh_features, embedded_mesh2mesh_edge_features,
           mesh2mesh_edge_indices, embedded_grid2mesh_edge_features,
           grid2mesh_edge_indices, embedded_mesh2grid_edge_features,
           mesh2grid_edge_indices, We1, be1, We2, be2, ln_e_scale, ln_e_bias,
           Wn1, bn1, Wn2, bn2, ln_n_scale, ln_n_bias):
    src = jnp.pad(mesh2mesh_edge_indices[0], (0, E_PAD - N_EDGES))
    dst = jnp.pad(mesh2mesh_edge_indices[1], (0, E_PAD - N_EDGES))
    dst2d = dst.reshape(E_PAD // _SK, _SK)
    nodes = jnp.pad(embedded_mesh_features, ((0, N_PAD - N_NODES), (0, 0)))
    edges_cm = jnp.pad(embedded_mesh2mesh_edge_features,
                       ((0, E_PAD - N_EDGES), (0, 0)))
    edges_cm = edges_cm.reshape(E_PAD, NCH, DC).transpose(1, 0, 2)

    r2 = lambda v: v.reshape(1, D)
    for l in range(NLAYER):
        w1a = We1[l, :D]
        w1b = We1[l, D:2 * D]
        w1c = We1[l, 2 * D:]
        p, q = _tc_proj(nodes, w1b, w1c)
        pack = lambda v: lax.bitcast_convert_type(
            v.reshape(N_PAD, D // 2, 2), jnp.int32)
        unpack = lambda v: lax.bitcast_convert_type(
            v, jnp.bfloat16).reshape(E_PAD, D)
        pg, qg = _sc_gather(pack(p), pack(q), src, dst)
        pg = unpack(pg)
        qg = unpack(qg)
        edges_cm = _tc_edge(edges_cm, pg, qg, w1a, r2(be1[l]), We2[l],
                            r2(be2[l]), r2(ln_e_scale[l]), r2(ln_e_bias[l]))
        agg_cm = _sc_scatter(edges_cm, dst2d)
        nodes = _tc_node(nodes, agg_cm, Wn1[l, :D], Wn1[l, D:], r2(bn1[l]),
                         Wn2[l], r2(bn2[l]), r2(ln_n_scale[l]),
                         r2(ln_n_bias[l]))

    nodes_out = nodes[:N_NODES]
    edges_out = edges_cm.transpose(1, 0, 2).reshape(E_PAD, D)[:N_EDGES]
    return (nodes_out, edges_out, embedded_grid2mesh_edge_features,
            embedded_mesh2grid_edge_features)


# speedup vs baseline: 3.5625x; 3.5625x over previous
import functools

import jax
import jax.numpy as jnp
from jax import lax
from jax.experimental import pallas as pl
from jax.experimental.pallas import tpu as pltpu
from jax.experimental.pallas import tpu_sc as plsc

N_NODES = 10242
N_EDGES = 81900
D = 512
DC = 128
NCH = D // DC
E_PAD = 81920
N_PAD = 10752
EB = 1024
NB = 512
NLAYER = 9


def _silu(x):
    return x * (1.0 / (1.0 + jnp.exp(-x)))


def _ln(h, s, b):
    mu = jnp.mean(h, axis=-1, keepdims=True)
    d = h - mu
    var = jnp.mean(d * d, axis=-1, keepdims=True)
    return d * lax.rsqrt(var + 1e-5) * s + b


def _pack_halves(x):
    lo = lax.bitcast_convert_type(x[:, :D // 2].astype(jnp.bfloat16),
                                  jnp.uint16).astype(jnp.int32)
    hi = lax.bitcast_convert_type(x[:, D // 2:].astype(jnp.bfloat16),
                                  jnp.uint16).astype(jnp.int32)
    return lo | (hi << 16)


def _unpack_halves(v):
    u = lax.bitcast_convert_type(v, jnp.uint32)
    lo = lax.bitcast_convert_type((u & 0xFFFF).astype(jnp.uint16),
                                  jnp.bfloat16).astype(jnp.float32)
    hi = lax.bitcast_convert_type((u >> 16).astype(jnp.uint16),
                                  jnp.bfloat16).astype(jnp.float32)
    return lo, hi


def _proj_body(nodes_ref, w1b_ref, w1c_ref, p_ref, q_ref):
    x = nodes_ref[...]
    p_ref[...] = _pack_halves(
        jnp.dot(x, w1b_ref[...], preferred_element_type=jnp.float32))
    q_ref[...] = _pack_halves(
        jnp.dot(x, w1c_ref[...], preferred_element_type=jnp.float32))


def _tc_proj(nodes, w1b, w1c):
    grid = (N_PAD // NB,)
    return pl.pallas_call(
        _proj_body,
        grid=grid,
        in_specs=[
            pl.BlockSpec((NB, D), lambda i: (i, 0)),
            pl.BlockSpec((D, D), lambda i: (0, 0)),
            pl.BlockSpec((D, D), lambda i: (0, 0)),
        ],
        out_specs=[
            pl.BlockSpec((NB, D // 2), lambda i: (i, 0)),
            pl.BlockSpec((NB, D // 2), lambda i: (i, 0)),
        ],
        out_shape=[
            jax.ShapeDtypeStruct((N_PAD, D // 2), jnp.int32),
            jax.ShapeDtypeStruct((N_PAD, D // 2), jnp.int32),
        ],
    )(nodes, w1b, w1c)


def _edge_body(e_ref, pg_ref, qg_ref, w1a_ref, b1_ref, w2_ref, b2_ref,
               s_ref, b_ref, out_ref):
    i = pl.program_id(0)
    e = jnp.concatenate([e_ref[c] for c in range(NCH)], axis=-1)
    x = jnp.dot(e, w1a_ref[...], preferred_element_type=jnp.float32)
    p_lo, p_hi = _unpack_halves(pg_ref[...])
    q_lo, q_hi = _unpack_halves(qg_ref[...])
    g = jnp.concatenate([p_lo + q_lo, p_hi + q_hi], axis=-1)
    x = x + g + b1_ref[...]
    h = _silu(x)
    y = jnp.dot(h, w2_ref[...], preferred_element_type=jnp.float32) + b2_ref[...]
    new = e + _ln(y, s_ref[...], b_ref[...])
    rows = i * EB + lax.broadcasted_iota(jnp.int32, (EB, 1), 0)
    new = jnp.where(rows < N_EDGES, new, 0.0)
    for c in range(NCH):
        out_ref[c] = new[:, c * DC:(c + 1) * DC]


def _tc_edge(edges_cm, pg, qg, w1a, b1, w2, b2, s, b):
    grid = (E_PAD // EB,)
    return pl.pallas_call(
        _edge_body,
        grid=grid,
        in_specs=[
            pl.BlockSpec((NCH, EB, DC), lambda i: (0, i, 0)),
            pl.BlockSpec((EB, D // 2), lambda i: (i, 0)),
            pl.BlockSpec((EB, D // 2), lambda i: (i, 0)),
            pl.BlockSpec((D, D), lambda i: (0, 0)),
            pl.BlockSpec((1, D), lambda i: (0, 0)),
            pl.BlockSpec((D, D), lambda i: (0, 0)),
            pl.BlockSpec((1, D), lambda i: (0, 0)),
            pl.BlockSpec((1, D), lambda i: (0, 0)),
            pl.BlockSpec((1, D), lambda i: (0, 0)),
        ],
        out_specs=pl.BlockSpec((NCH, EB, DC), lambda i: (0, i, 0)),
        out_shape=jax.ShapeDtypeStruct((NCH, E_PAD, DC), jnp.float32),
    )(edges_cm, pg, qg, w1a, b1, w2, b2, s, b)


def _node_body(n_ref, agg_ref, wn1a_ref, wn1b_ref, b1_ref, wn2_ref, b2_ref,
               s_ref, b_ref, out_ref):
    n = n_ref[...]
    agg = jnp.concatenate([agg_ref[c] for c in range(NCH)], axis=-1)
    x = (jnp.dot(n, wn1a_ref[...], preferred_element_type=jnp.float32)
         + jnp.dot(agg, wn1b_ref[...], preferred_element_type=jnp.float32)
         + b1_ref[...])
    h = _silu(x)
    y = jnp.dot(h, wn2_ref[...], preferred_element_type=jnp.float32) + b2_ref[...]
    out_ref[...] = n + _ln(y, s_ref[...], b_ref[...])


def _tc_node(nodes, agg_cm, wn1a, wn1b, b1, wn2, b2, s, b):
    grid = (N_PAD // NB,)
    return pl.pallas_call(
        _node_body,
        grid=grid,
        in_specs=[
            pl.BlockSpec((NB, D), lambda i: (i, 0)),
            pl.BlockSpec((NCH, NB, DC), lambda i: (0, i, 0)),
            pl.BlockSpec((D, D), lambda i: (0, 0)),
            pl.BlockSpec((D, D), lambda i: (0, 0)),
            pl.BlockSpec((1, D), lambda i: (0, 0)),
            pl.BlockSpec((D, D), lambda i: (0, 0)),
            pl.BlockSpec((1, D), lambda i: (0, 0)),
            pl.BlockSpec((1, D), lambda i: (0, 0)),
            pl.BlockSpec((1, D), lambda i: (0, 0)),
        ],
        out_specs=pl.BlockSpec((NB, D), lambda i: (i, 0)),
        out_shape=jax.ShapeDtypeStruct((N_PAD, D), jnp.float32),
    )(nodes, agg_cm, wn1a, wn1b, b1, wn2, b2, s, b)


_NW = 32
_GK = 40
_SK = 128
_SLAB = N_PAD // 16
_SB = 96


def _gather_body(p_hbm, q_hbm, src_hbm, dst_hbm, pg_hbm, qg_hbm,
                 sidx_v, didx_v, pb0, pb1, qb0, qb1,
                 sp0, sp1, sq0, sq1):
    per_w = E_PAD // _NW
    n_iter = per_w // _GK // 2
    wid = lax.axis_index("s") * 2 + lax.axis_index("c")
    base0 = wid * per_w

    pltpu.sync_copy(src_hbm.at[pl.ds(base0, per_w)], sidx_v)
    pltpu.sync_copy(dst_hbm.at[pl.ds(base0, per_w)], didx_v)

    pbufs, qbufs = (pb0, pb1), (qb0, qb1)
    psems, qsems = (sp0, sp1), (sq0, sq1)

    def body(i, carry):
        for b in range(2):
            c = 2 * i + b
            off = c * _GK
            @pl.when(i > 0)
            def _():
                pltpu.make_async_copy(
                    pbufs[b], pg_hbm.at[pl.ds(0, _GK)], psems[b]).wait()
                pltpu.make_async_copy(
                    qbufs[b], qg_hbm.at[pl.ds(0, _GK)], qsems[b]).wait()
            pltpu.async_copy(p_hbm.at[sidx_v.at[pl.ds(off, _GK)]],
                             pbufs[b], psems[b])
            pltpu.async_copy(q_hbm.at[didx_v.at[pl.ds(off, _GK)]],
                             qbufs[b], qsems[b])
        for b in range(2):
            c = 2 * i + b
            base = base0 + c * _GK
            pltpu.make_async_copy(p_hbm.at[pl.ds(0, _GK)], pbufs[b],
                                  psems[b]).wait()
            pltpu.async_copy(pbufs[b], pg_hbm.at[pl.ds(base, _GK)], psems[b])
            pltpu.make_async_copy(q_hbm.at[pl.ds(0, _GK)], qbufs[b],
                                  qsems[b]).wait()
            pltpu.async_copy(qbufs[b], qg_hbm.at[pl.ds(base, _GK)], qsems[b])
        return carry

    lax.fori_loop(0, n_iter, body, 0)
    for b in range(2):
        pltpu.make_async_copy(pbufs[b], pg_hbm.at[pl.ds(0, _GK)],
                              psems[b]).wait()
        pltpu.make_async_copy(qbufs[b], qg_hbm.at[pl.ds(0, _GK)],
                              qsems[b]).wait()


def _sc_gather(p, q, src, dst):
    mesh = plsc.VectorSubcoreMesh(core_axis_name="c", subcore_axis_name="s")
    per_w = E_PAD // _NW
    fn = pl.kernel(
        _gather_body,
        out_type=[
            jax.ShapeDtypeStruct((E_PAD, D // 2), jnp.int32),
            jax.ShapeDtypeStruct((E_PAD, D // 2), jnp.int32),
        ],
        mesh=mesh,
        scratch_types=[
            pltpu.VMEM((per_w,), jnp.int32),
            pltpu.VMEM((per_w,), jnp.int32),
            pltpu.VMEM((_GK, D // 2), jnp.int32),
            pltpu.VMEM((_GK, D // 2), jnp.int32),
            pltpu.VMEM((_GK, D // 2), jnp.int32),
            pltpu.VMEM((_GK, D // 2), jnp.int32),
            pltpu.SemaphoreType.DMA,
            pltpu.SemaphoreType.DMA,
            pltpu.SemaphoreType.DMA,
            pltpu.SemaphoreType.DMA,
        ],
    )
    return fn(p, q, src, dst)


def _scatter_body(e_hbm, dst2d_hbm, agg_hbm, r0_v, r1_v, idx_v, spmem,
                  sa0, sa1, sd0, sd1):
    cid = lax.axis_index("c")
    sid = lax.axis_index("s")
    per_tile = E_PAD // 16
    n_iter = per_tile // _SK
    n_sl = _SLAB // _SB

    rows = (r0_v, r1_v)
    asems = (sa0, sa1)
    dsems = (sd0, sd1)

    z16 = jnp.zeros((16,), jnp.float32)

    def zrow(i, carry):
        for j in range(DC // 16):
            r0_v[i, pl.ds(j * 16, 16)] = z16
        return carry

    pltpu.sync_copy(dst2d_hbm.at[pl.ds(sid * n_iter, n_iter)], idx_v)

    for j in range(2):
        chunk = cid * 2 + j

        lax.fori_loop(0, _SK, zrow, 0)
        base_r = sid * _SLAB
        for t in range(_SLAB // _SK):
            pltpu.sync_copy(r0_v, spmem.at[pl.ds(base_r + t * _SK, _SK), :])
        rem = _SLAB % _SK
        if rem:
            pltpu.sync_copy(r0_v.at[pl.ds(0, rem)],
                            spmem.at[pl.ds(base_r + _SLAB - rem, rem), :])
        plsc.subcore_barrier()

        def body(i, carry):
            for b in range(2):
                c = 2 * i + b
                base = sid * per_tile + c * _SK

                @pl.when(i > 0)
                def _():
                    pltpu.make_async_copy(
                        rows[b], spmem.at[pl.ds(0, _SK)], asems[b]).wait()

                pltpu.sync_copy(e_hbm.at[chunk, pl.ds(base, _SK)], rows[b])
                pltpu.async_copy(rows[b], spmem.at[idx_v.at[c]], asems[b],
                                 add=True)
            return carry

        lax.fori_loop(0, n_iter // 2, body, 0)
        for b in range(2):
            pltpu.make_async_copy(rows[b], spmem.at[pl.ds(0, _SK)],
                                  asems[b]).wait()
        plsc.subcore_barrier()

        def drain(t, carry):
            for b in range(2):
                tt = 2 * t + b
                r0 = sid * _SLAB + tt * _SB

                @pl.when(tt < n_sl)
                def _():
                    @pl.when(t > 0)
                    def _():
                        pltpu.make_async_copy(
                            rows[b].at[pl.ds(0, _SB)],
                            agg_hbm.at[chunk, pl.ds(0, _SB)], dsems[b]).wait()
                    pltpu.sync_copy(spmem.at[pl.ds(r0, _SB), :],
                                    rows[b].at[pl.ds(0, _SB)])
                    pltpu.async_copy(rows[b].at[pl.ds(0, _SB)],
                                     agg_hbm.at[chunk, pl.ds(r0, _SB)],
                                     dsems[b])
            return carry

        lax.fori_loop(0, (n_sl + 1) // 2, drain, 0)
        for b in range(2):
            pltpu.make_async_copy(rows[b].at[pl.ds(0, _SB)],
                                  agg_hbm.at[chunk, pl.ds(0, _SB)],
                                  dsems[b]).wait()
        plsc.subcore_barrier()


def _sc_scatter(edges_cm, dst2d):
    mesh = plsc.VectorSubcoreMesh(core_axis_name="c", subcore_axis_name="s")
    fn = pl.kernel(
        _scatter_body,
        out_type=jax.ShapeDtypeStruct((NCH, N_PAD, DC), jnp.float32),
        mesh=mesh,
        scratch_types=[
            pltpu.VMEM((_SK, DC), jnp.float32),
            pltpu.VMEM((_SK, DC), jnp.float32),
            pltpu.VMEM((E_PAD // 16 // _SK, _SK), jnp.int32),
            pltpu.VMEM_SHARED((N_PAD, DC), jnp.float32),
            pltpu.SemaphoreType.DMA,
            pltpu.SemaphoreType.DMA,
            pltpu.SemaphoreType.DMA,
            pltpu.SemaphoreType.DMA,
        ],
    )
    return fn(edges_cm, dst2d)


def kernel(embedded_mesh_features, embedded_mesh2mesh_edge_features,
           mesh2mesh_edge_indices, embedded_grid2mesh_edge_features,
           grid2mesh_edge_indices, embedded_mesh2grid_edge_features,
           mesh2grid_edge_indices, We1, be1, We2, be2, ln_e_scale, ln_e_bias,
           Wn1, bn1, Wn2, bn2, ln_n_scale, ln_n_bias):
    src = jnp.pad(mesh2mesh_edge_indices[0], (0, E_PAD - N_EDGES))
    dst = jnp.pad(mesh2mesh_edge_indices[1], (0, E_PAD - N_EDGES))
    dst2d = dst.reshape(E_PAD // _SK, _SK)
    nodes = jnp.pad(embedded_mesh_features, ((0, N_PAD - N_NODES), (0, 0)))
    edges_cm = jnp.pad(embedded_mesh2mesh_edge_features,
                       ((0, E_PAD - N_EDGES), (0, 0)))
    edges_cm = edges_cm.reshape(E_PAD, NCH, DC).transpose(1, 0, 2)

    r2 = lambda v: v.reshape(1, D)
    for l in range(NLAYER):
        w1a = We1[l, :D]
        w1b = We1[l, D:2 * D]
        w1c = We1[l, 2 * D:]
        p, q = _tc_proj(nodes, w1b, w1c)
        pg, qg = _sc_gather(p, q, src, dst)
        edges_cm = _tc_edge(edges_cm, pg, qg, w1a, r2(be1[l]), We2[l],
                            r2(be2[l]), r2(ln_e_scale[l]), r2(ln_e_bias[l]))
        agg_cm = _sc_scatter(edges_cm, dst2d)
        nodes = _tc_node(nodes, agg_cm, Wn1[l, :D], Wn1[l, D:], r2(bn1[l]),
                         Wn2[l], r2(bn2[l]), r2(ln_n_scale[l]),
                         r2(ln_n_bias[l]))

    nodes_out = nodes[:N_NODES]
    edges_out = edges_cm.transpose(1, 0, 2).reshape(E_PAD, D)[:N_EDGES]
    return (nodes_out, edges_out, embedded_grid2mesh_edge_features,
            embedded_mesh2grid_edge_features)


# speedup vs baseline: 3.9516x; 1.1092x over previous
import functools

import jax
import jax.numpy as jnp
from jax import lax
from jax.experimental import pallas as pl
from jax.experimental.pallas import tpu as pltpu
from jax.experimental.pallas import tpu_sc as plsc

N_NODES = 10242
N_EDGES = 81900
D = 512
DC = 128
NCH = D // DC
E_PAD = 81920
N_PAD = 10752
EB = 1024
NB = 512
NLAYER = 9


def _silu(x):
    return x * (1.0 / (1.0 + jnp.exp(-x)))


def _ln(h, s, b):
    mu = jnp.mean(h, axis=-1, keepdims=True)
    d = h - mu
    var = jnp.mean(d * d, axis=-1, keepdims=True)
    return d * lax.rsqrt(var + 1e-5) * s + b


def _pack_halves(x):
    lo = lax.bitcast_convert_type(x[:, :D // 2].astype(jnp.bfloat16),
                                  jnp.uint16).astype(jnp.int32)
    hi = lax.bitcast_convert_type(x[:, D // 2:].astype(jnp.bfloat16),
                                  jnp.uint16).astype(jnp.int32)
    return lo | (hi << 16)


def _unpack_halves(v):
    u = lax.bitcast_convert_type(v, jnp.uint32)
    lo = lax.bitcast_convert_type((u & 0xFFFF).astype(jnp.uint16),
                                  jnp.bfloat16).astype(jnp.float32)
    hi = lax.bitcast_convert_type((u >> 16).astype(jnp.uint16),
                                  jnp.bfloat16).astype(jnp.float32)
    return lo, hi


def _proj_body(nodes_ref, w1b_ref, w1c_ref, p_ref, q_ref):
    x = nodes_ref[...]
    p_ref[...] = _pack_halves(
        jnp.dot(x, w1b_ref[...], preferred_element_type=jnp.float32))
    q_ref[...] = _pack_halves(
        jnp.dot(x, w1c_ref[...], preferred_element_type=jnp.float32))


def _tc_proj(nodes, w1b, w1c):
    grid = (N_PAD // NB,)
    return pl.pallas_call(
        _proj_body,
        grid=grid,
        in_specs=[
            pl.BlockSpec((NB, D), lambda i: (i, 0)),
            pl.BlockSpec((D, D), lambda i: (0, 0)),
            pl.BlockSpec((D, D), lambda i: (0, 0)),
        ],
        out_specs=[
            pl.BlockSpec((NB, D // 2), lambda i: (i, 0)),
            pl.BlockSpec((NB, D // 2), lambda i: (i, 0)),
        ],
        out_shape=[
            jax.ShapeDtypeStruct((N_PAD, D // 2), jnp.int32),
            jax.ShapeDtypeStruct((N_PAD, D // 2), jnp.int32),
        ],
    )(nodes, w1b, w1c)


def _edge_body(e_ref, pg_ref, qg_ref, w1a_ref, b1_ref, w2_ref, b2_ref,
               s_ref, b_ref, out_ref):
    i = pl.program_id(0)
    e = e_ref[...]
    x = jnp.dot(e, w1a_ref[...], preferred_element_type=jnp.float32)
    p_lo, p_hi = _unpack_halves(pg_ref[...])
    q_lo, q_hi = _unpack_halves(qg_ref[...])
    g = jnp.concatenate([p_lo + q_lo, p_hi + q_hi], axis=-1)
    x = x + g + b1_ref[...]
    h = _silu(x)
    y = jnp.dot(h, w2_ref[...], preferred_element_type=jnp.float32) + b2_ref[...]
    new = e + _ln(y, s_ref[...], b_ref[...])
    rows = i * EB + lax.broadcasted_iota(jnp.int32, (EB, 1), 0)
    out_ref[...] = jnp.where(rows < N_EDGES, new, 0.0)


def _tc_edge(edges, pg, qg, w1a, b1, w2, b2, s, b):
    grid = (E_PAD // EB,)
    return pl.pallas_call(
        _edge_body,
        grid=grid,
        in_specs=[
            pl.BlockSpec((EB, D), lambda i: (i, 0)),
            pl.BlockSpec((EB, D // 2), lambda i: (i, 0)),
            pl.BlockSpec((EB, D // 2), lambda i: (i, 0)),
            pl.BlockSpec((D, D), lambda i: (0, 0)),
            pl.BlockSpec((1, D), lambda i: (0, 0)),
            pl.BlockSpec((D, D), lambda i: (0, 0)),
            pl.BlockSpec((1, D), lambda i: (0, 0)),
            pl.BlockSpec((1, D), lambda i: (0, 0)),
            pl.BlockSpec((1, D), lambda i: (0, 0)),
        ],
        out_specs=pl.BlockSpec((EB, D), lambda i: (i, 0)),
        out_shape=jax.ShapeDtypeStruct((E_PAD, D), jnp.float32),
    )(edges, pg, qg, w1a, b1, w2, b2, s, b)


def _node_body(n_ref, agg_ref, wn1a_ref, wn1b_ref, b1_ref, wn2_ref, b2_ref,
               s_ref, b_ref, out_ref):
    n = n_ref[...]
    agg = agg_ref[...]
    x = (jnp.dot(n, wn1a_ref[...], preferred_element_type=jnp.float32)
         + jnp.dot(agg, wn1b_ref[...], preferred_element_type=jnp.float32)
         + b1_ref[...])
    h = _silu(x)
    y = jnp.dot(h, wn2_ref[...], preferred_element_type=jnp.float32) + b2_ref[...]
    out_ref[...] = n + _ln(y, s_ref[...], b_ref[...])


def _tc_node(nodes, agg_cm, wn1a, wn1b, b1, wn2, b2, s, b):
    grid = (N_PAD // NB,)
    return pl.pallas_call(
        _node_body,
        grid=grid,
        in_specs=[
            pl.BlockSpec((NB, D), lambda i: (i, 0)),
            pl.BlockSpec((NB, D), lambda i: (i, 0)),
            pl.BlockSpec((D, D), lambda i: (0, 0)),
            pl.BlockSpec((D, D), lambda i: (0, 0)),
            pl.BlockSpec((1, D), lambda i: (0, 0)),
            pl.BlockSpec((D, D), lambda i: (0, 0)),
            pl.BlockSpec((1, D), lambda i: (0, 0)),
            pl.BlockSpec((1, D), lambda i: (0, 0)),
            pl.BlockSpec((1, D), lambda i: (0, 0)),
        ],
        out_specs=pl.BlockSpec((NB, D), lambda i: (i, 0)),
        out_shape=jax.ShapeDtypeStruct((N_PAD, D), jnp.float32),
    )(nodes, agg_cm, wn1a, wn1b, b1, wn2, b2, s, b)


_NW = 32
_GK = 40
_SK = 128
_SLAB = N_PAD // 16
_SB = 96


def _gather_body(p_hbm, q_hbm, src_hbm, dst_hbm, pg_hbm, qg_hbm,
                 sidx_v, didx_v, pb0, pb1, qb0, qb1,
                 sp0, sp1, sq0, sq1):
    per_w = E_PAD // _NW
    n_iter = per_w // _GK // 2
    wid = lax.axis_index("s") * 2 + lax.axis_index("c")
    base0 = wid * per_w

    pltpu.sync_copy(src_hbm.at[pl.ds(base0, per_w)], sidx_v)
    pltpu.sync_copy(dst_hbm.at[pl.ds(base0, per_w)], didx_v)

    pbufs, qbufs = (pb0, pb1), (qb0, qb1)
    psems, qsems = (sp0, sp1), (sq0, sq1)

    def body(i, carry):
        for b in range(2):
            c = 2 * i + b
            off = c * _GK
            @pl.when(i > 0)
            def _():
                pltpu.make_async_copy(
                    pbufs[b], pg_hbm.at[pl.ds(0, _GK)], psems[b]).wait()
                pltpu.make_async_copy(
                    qbufs[b], qg_hbm.at[pl.ds(0, _GK)], qsems[b]).wait()
            pltpu.async_copy(p_hbm.at[sidx_v.at[pl.ds(off, _GK)]],
                             pbufs[b], psems[b])
            pltpu.async_copy(q_hbm.at[didx_v.at[pl.ds(off, _GK)]],
                             qbufs[b], qsems[b])
        for b in range(2):
            c = 2 * i + b
            base = base0 + c * _GK
            pltpu.make_async_copy(p_hbm.at[pl.ds(0, _GK)], pbufs[b],
                                  psems[b]).wait()
            pltpu.async_copy(pbufs[b], pg_hbm.at[pl.ds(base, _GK)], psems[b])
            pltpu.make_async_copy(q_hbm.at[pl.ds(0, _GK)], qbufs[b],
                                  qsems[b]).wait()
            pltpu.async_copy(qbufs[b], qg_hbm.at[pl.ds(base, _GK)], qsems[b])
        return carry

    lax.fori_loop(0, n_iter, body, 0)
    for b in range(2):
        pltpu.make_async_copy(pbufs[b], pg_hbm.at[pl.ds(0, _GK)],
                              psems[b]).wait()
        pltpu.make_async_copy(qbufs[b], qg_hbm.at[pl.ds(0, _GK)],
                              qsems[b]).wait()


def _sc_gather(p, q, src, dst):
    mesh = plsc.VectorSubcoreMesh(core_axis_name="c", subcore_axis_name="s")
    per_w = E_PAD // _NW
    fn = pl.kernel(
        _gather_body,
        out_type=[
            jax.ShapeDtypeStruct((E_PAD, D // 2), jnp.int32),
            jax.ShapeDtypeStruct((E_PAD, D // 2), jnp.int32),
        ],
        mesh=mesh,
        scratch_types=[
            pltpu.VMEM((per_w,), jnp.int32),
            pltpu.VMEM((per_w,), jnp.int32),
            pltpu.VMEM((_GK, D // 2), jnp.int32),
            pltpu.VMEM((_GK, D // 2), jnp.int32),
            pltpu.VMEM((_GK, D // 2), jnp.int32),
            pltpu.VMEM((_GK, D // 2), jnp.int32),
            pltpu.SemaphoreType.DMA,
            pltpu.SemaphoreType.DMA,
            pltpu.SemaphoreType.DMA,
            pltpu.SemaphoreType.DMA,
        ],
    )
    return fn(p, q, src, dst)


def _scatter_body(e_hbm, dst2d_hbm, agg_hbm, r0_v, r1_v, idx_v, spmem,
                  sa0, sa1, sd0, sd1):
    cid = lax.axis_index("c")
    sid = lax.axis_index("s")
    per_tile = E_PAD // 16
    n_iter = per_tile // _SK
    n_sl = _SLAB // _SB

    rows = (r0_v, r1_v)
    asems = (sa0, sa1)
    dsems = (sd0, sd1)

    z16 = jnp.zeros((16,), jnp.float32)

    def zrow(i, carry):
        for j in range(DC // 16):
            r0_v[i, pl.ds(j * 16, 16)] = z16
        return carry

    pltpu.sync_copy(dst2d_hbm.at[pl.ds(sid * n_iter, n_iter)], idx_v)

    for j in range(2):
        chunk = cid * 2 + j

        lax.fori_loop(0, _SK, zrow, 0)
        base_r = sid * _SLAB
        for t in range(_SLAB // _SK):
            pltpu.sync_copy(r0_v, spmem.at[pl.ds(base_r + t * _SK, _SK), :])
        rem = _SLAB % _SK
        if rem:
            pltpu.sync_copy(r0_v.at[pl.ds(0, rem)],
                            spmem.at[pl.ds(base_r + _SLAB - rem, rem), :])
        plsc.subcore_barrier()

        def body(i, carry):
            for b in range(2):
                c = 2 * i + b
                base = sid * per_tile + c * _SK

                @pl.when(i > 0)
                def _():
                    pltpu.make_async_copy(
                        rows[b], spmem.at[pl.ds(0, _SK)], asems[b]).wait()

                pltpu.sync_copy(e_hbm.at[pl.ds(base, _SK), pl.ds(chunk * DC, DC)],
                                rows[b])
                pltpu.async_copy(rows[b], spmem.at[idx_v.at[c]], asems[b],
                                 add=True)
            return carry

        lax.fori_loop(0, n_iter // 2, body, 0)
        for b in range(2):
            pltpu.make_async_copy(rows[b], spmem.at[pl.ds(0, _SK)],
                                  asems[b]).wait()
        plsc.subcore_barrier()

        def drain(t, carry):
            for b in range(2):
                tt = 2 * t + b
                r0 = sid * _SLAB + tt * _SB

                @pl.when(tt < n_sl)
                def _():
                    @pl.when(t > 0)
                    def _():
                        pltpu.make_async_copy(
                            rows[b].at[pl.ds(0, _SB)],
                            agg_hbm.at[pl.ds(0, _SB), pl.ds(0, DC)],
                            dsems[b]).wait()
                    pltpu.sync_copy(spmem.at[pl.ds(r0, _SB), :],
                                    rows[b].at[pl.ds(0, _SB)])
                    pltpu.async_copy(rows[b].at[pl.ds(0, _SB)],
                                     agg_hbm.at[pl.ds(r0, _SB),
                                                pl.ds(chunk * DC, DC)],
                                     dsems[b])
            return carry

        lax.fori_loop(0, (n_sl + 1) // 2, drain, 0)
        for b in range(2):
            pltpu.make_async_copy(rows[b].at[pl.ds(0, _SB)],
                                  agg_hbm.at[pl.ds(0, _SB), pl.ds(0, DC)],
                                  dsems[b]).wait()
        plsc.subcore_barrier()


def _sc_scatter(edges_cm, dst2d):
    mesh = plsc.VectorSubcoreMesh(core_axis_name="c", subcore_axis_name="s")
    fn = pl.kernel(
        _scatter_body,
        out_type=jax.ShapeDtypeStruct((N_PAD, D), jnp.float32),
        mesh=mesh,
        scratch_types=[
            pltpu.VMEM((_SK, DC), jnp.float32),
            pltpu.VMEM((_SK, DC), jnp.float32),
            pltpu.VMEM((E_PAD // 16 // _SK, _SK), jnp.int32),
            pltpu.VMEM_SHARED((N_PAD, DC), jnp.float32),
            pltpu.SemaphoreType.DMA,
            pltpu.SemaphoreType.DMA,
            pltpu.SemaphoreType.DMA,
            pltpu.SemaphoreType.DMA,
        ],
    )
    return fn(edges_cm, dst2d)


def kernel(embedded_mesh_features, embedded_mesh2mesh_edge_features,
           mesh2mesh_edge_indices, embedded_grid2mesh_edge_features,
           grid2mesh_edge_indices, embedded_mesh2grid_edge_features,
           mesh2grid_edge_indices, We1, be1, We2, be2, ln_e_scale, ln_e_bias,
           Wn1, bn1, Wn2, bn2, ln_n_scale, ln_n_bias):
    src = jnp.pad(mesh2mesh_edge_indices[0], (0, E_PAD - N_EDGES))
    dst = jnp.pad(mesh2mesh_edge_indices[1], (0, E_PAD - N_EDGES))
    dst2d = dst.reshape(E_PAD // _SK, _SK)
    nodes = jnp.pad(embedded_mesh_features, ((0, N_PAD - N_NODES), (0, 0)))
    edges = embedded_mesh2mesh_edge_features

    r2 = lambda v: v.reshape(1, D)
    for l in range(NLAYER):
        w1a = We1[l, :D]
        w1b = We1[l, D:2 * D]
        w1c = We1[l, 2 * D:]
        p, q = _tc_proj(nodes, w1b, w1c)
        pg, qg = _sc_gather(p, q, src, dst)
        edges = _tc_edge(edges, pg, qg, w1a, r2(be1[l]), We2[l],
                         r2(be2[l]), r2(ln_e_scale[l]), r2(ln_e_bias[l]))
        agg = _sc_scatter(edges, dst2d)
        nodes = _tc_node(nodes, agg, Wn1[l, :D], Wn1[l, D:], r2(bn1[l]),
                         Wn2[l], r2(bn2[l]), r2(ln_n_scale[l]),
                         r2(ln_n_bias[l]))

    nodes_out = nodes[:N_NODES]
    edges_out = edges[:N_EDGES]
    return (nodes_out, edges_out, embedded_grid2mesh_edge_features,
            embedded_mesh2grid_edge_features)


# speedup vs baseline: 4.2326x; 1.0711x over previous
import functools

import jax
import jax.numpy as jnp
from jax import lax
from jax.experimental import pallas as pl
from jax.experimental.pallas import tpu as pltpu
from jax.experimental.pallas import tpu_sc as plsc

N_NODES = 10242
N_EDGES = 81900
D = 512
DC = 128
NCH = D // DC
E_PAD = 81920
N_PAD = 10752
EB = 1024
NB = 512
NLAYER = 9


def _silu(x):
    return x * (1.0 / (1.0 + jnp.exp(-x)))


def _ln(h, s, b):
    mu = jnp.mean(h, axis=-1, keepdims=True)
    d = h - mu
    var = jnp.mean(d * d, axis=-1, keepdims=True)
    return d * lax.rsqrt(var + 1e-5) * s + b


def _pack_halves(x):
    lo = lax.bitcast_convert_type(x[:, :D // 2].astype(jnp.bfloat16),
                                  jnp.uint16).astype(jnp.int32)
    hi = lax.bitcast_convert_type(x[:, D // 2:].astype(jnp.bfloat16),
                                  jnp.uint16).astype(jnp.int32)
    return lo | (hi << 16)


def _unpack_halves(v):
    u = lax.bitcast_convert_type(v, jnp.uint32)
    lo = lax.bitcast_convert_type((u & 0xFFFF).astype(jnp.uint16),
                                  jnp.bfloat16).astype(jnp.float32)
    hi = lax.bitcast_convert_type((u >> 16).astype(jnp.uint16),
                                  jnp.bfloat16).astype(jnp.float32)
    return lo, hi


def _proj_body(nodes_ref, w1b_ref, w1c_ref, p_ref, q_ref):
    x = nodes_ref[...]
    p_ref[...] = _pack_halves(
        jnp.dot(x, w1b_ref[...], preferred_element_type=jnp.float32))
    q_ref[...] = _pack_halves(
        jnp.dot(x, w1c_ref[...], preferred_element_type=jnp.float32))


def _tc_proj(nodes, w1b, w1c):
    grid = (N_PAD // NB,)
    return pl.pallas_call(
        _proj_body,
        grid=grid,
        in_specs=[
            pl.BlockSpec((NB, D), lambda i: (i, 0)),
            pl.BlockSpec((D, D), lambda i: (0, 0)),
            pl.BlockSpec((D, D), lambda i: (0, 0)),
        ],
        out_specs=[
            pl.BlockSpec((NB, D // 2), lambda i: (i, 0)),
            pl.BlockSpec((NB, D // 2), lambda i: (i, 0)),
        ],
        out_shape=[
            jax.ShapeDtypeStruct((N_PAD, D // 2), jnp.int32),
            jax.ShapeDtypeStruct((N_PAD, D // 2), jnp.int32),
        ],
    )(nodes, w1b, w1c)


def _edge_body(row_off, e_ref, pg_ref, qg_ref, w1a_ref, b1_ref, w2_ref,
               b2_ref, s_ref, b_ref, out_ref):
    i = pl.program_id(0)
    e = e_ref[...]
    x = jnp.dot(e, w1a_ref[...], preferred_element_type=jnp.float32)
    p_lo, p_hi = _unpack_halves(pg_ref[...])
    q_lo, q_hi = _unpack_halves(qg_ref[...])
    g = jnp.concatenate([p_lo + q_lo, p_hi + q_hi], axis=-1)
    x = x + g + b1_ref[...]
    h = _silu(x)
    y = jnp.dot(h, w2_ref[...], preferred_element_type=jnp.float32) + b2_ref[...]
    new = e + _ln(y, s_ref[...], b_ref[...])
    rows = row_off + i * EB + lax.broadcasted_iota(jnp.int32, (EB, 1), 0)
    out_ref[...] = jnp.where(rows < N_EDGES, new, 0.0)


def _tc_edge(edges, pg, qg, w1a, b1, w2, b2, s, b, row_off, e_len,
             in_off_blocks):
    grid = (e_len // EB,)
    return pl.pallas_call(
        functools.partial(_edge_body, row_off),
        grid=grid,
        in_specs=[
            pl.BlockSpec((EB, D), lambda i: (i + in_off_blocks, 0)),
            pl.BlockSpec((EB, D // 2), lambda i: (i, 0)),
            pl.BlockSpec((EB, D // 2), lambda i: (i, 0)),
            pl.BlockSpec((D, D), lambda i: (0, 0)),
            pl.BlockSpec((1, D), lambda i: (0, 0)),
            pl.BlockSpec((D, D), lambda i: (0, 0)),
            pl.BlockSpec((1, D), lambda i: (0, 0)),
            pl.BlockSpec((1, D), lambda i: (0, 0)),
            pl.BlockSpec((1, D), lambda i: (0, 0)),
        ],
        out_specs=pl.BlockSpec((EB, D), lambda i: (i, 0)),
        out_shape=jax.ShapeDtypeStruct((e_len, D), jnp.float32),
    )(edges, pg, qg, w1a, b1, w2, b2, s, b)


def _node_body(n_ref, agg0_ref, agg1_ref, wn1a_ref, wn1b_ref, b1_ref,
               wn2_ref, b2_ref, s_ref, b_ref, out_ref):
    n = n_ref[...]
    agg = agg0_ref[...] + agg1_ref[...]
    x = (jnp.dot(n, wn1a_ref[...], preferred_element_type=jnp.float32)
         + jnp.dot(agg, wn1b_ref[...], preferred_element_type=jnp.float32)
         + b1_ref[...])
    h = _silu(x)
    y = jnp.dot(h, wn2_ref[...], preferred_element_type=jnp.float32) + b2_ref[...]
    out_ref[...] = n + _ln(y, s_ref[...], b_ref[...])


def _tc_node(nodes, agg0, agg1, wn1a, wn1b, b1, wn2, b2, s, b):
    grid = (N_PAD // NB,)
    return pl.pallas_call(
        _node_body,
        grid=grid,
        in_specs=[
            pl.BlockSpec((NB, D), lambda i: (i, 0)),
            pl.BlockSpec((NB, D), lambda i: (i, 0)),
            pl.BlockSpec((NB, D), lambda i: (i, 0)),
            pl.BlockSpec((D, D), lambda i: (0, 0)),
            pl.BlockSpec((D, D), lambda i: (0, 0)),
            pl.BlockSpec((1, D), lambda i: (0, 0)),
            pl.BlockSpec((D, D), lambda i: (0, 0)),
            pl.BlockSpec((1, D), lambda i: (0, 0)),
            pl.BlockSpec((1, D), lambda i: (0, 0)),
            pl.BlockSpec((1, D), lambda i: (0, 0)),
        ],
        out_specs=pl.BlockSpec((NB, D), lambda i: (i, 0)),
        out_shape=jax.ShapeDtypeStruct((N_PAD, D), jnp.float32),
    )(nodes, agg0, agg1, wn1a, wn1b, b1, wn2, b2, s, b)


_NW = 32
_GK = 40
_SK = 128
_SLAB = N_PAD // 16
_SB = 96


def _gather_body(e_off, e_len, p_hbm, q_hbm, src_hbm, dst_hbm, pg_hbm, qg_hbm,
                 sidx_v, didx_v, pb0, pb1, qb0, qb1,
                 sp0, sp1, sq0, sq1):
    per_w = e_len // _NW
    n_iter = per_w // _GK // 2
    wid = lax.axis_index("s") * 2 + lax.axis_index("c")
    base0 = wid * per_w

    pltpu.sync_copy(src_hbm.at[pl.ds(e_off + base0, per_w)], sidx_v)
    pltpu.sync_copy(dst_hbm.at[pl.ds(e_off + base0, per_w)], didx_v)

    pbufs, qbufs = (pb0, pb1), (qb0, qb1)
    psems, qsems = (sp0, sp1), (sq0, sq1)

    def body(i, carry):
        for b in range(2):
            c = 2 * i + b
            off = c * _GK
            @pl.when(i > 0)
            def _():
                pltpu.make_async_copy(
                    pbufs[b], pg_hbm.at[pl.ds(0, _GK)], psems[b]).wait()
                pltpu.make_async_copy(
                    qbufs[b], qg_hbm.at[pl.ds(0, _GK)], qsems[b]).wait()
            pltpu.async_copy(p_hbm.at[sidx_v.at[pl.ds(off, _GK)]],
                             pbufs[b], psems[b])
            pltpu.async_copy(q_hbm.at[didx_v.at[pl.ds(off, _GK)]],
                             qbufs[b], qsems[b])
        for b in range(2):
            c = 2 * i + b
            base = base0 + c * _GK
            pltpu.make_async_copy(p_hbm.at[pl.ds(0, _GK)], pbufs[b],
                                  psems[b]).wait()
            pltpu.async_copy(pbufs[b], pg_hbm.at[pl.ds(base, _GK)], psems[b])
            pltpu.make_async_copy(q_hbm.at[pl.ds(0, _GK)], qbufs[b],
                                  qsems[b]).wait()
            pltpu.async_copy(qbufs[b], qg_hbm.at[pl.ds(base, _GK)], qsems[b])
        return carry

    lax.fori_loop(0, n_iter, body, 0)
    for b in range(2):
        pltpu.make_async_copy(pbufs[b], pg_hbm.at[pl.ds(0, _GK)],
                              psems[b]).wait()
        pltpu.make_async_copy(qbufs[b], qg_hbm.at[pl.ds(0, _GK)],
                              qsems[b]).wait()


def _sc_gather(p, q, src, dst, e_off, e_len):
    mesh = plsc.VectorSubcoreMesh(core_axis_name="c", subcore_axis_name="s")
    per_w = e_len // _NW
    fn = pl.kernel(
        functools.partial(_gather_body, e_off, e_len),
        out_type=[
            jax.ShapeDtypeStruct((e_len, D // 2), jnp.int32),
            jax.ShapeDtypeStruct((e_len, D // 2), jnp.int32),
        ],
        mesh=mesh,
        scratch_types=[
            pltpu.VMEM((per_w,), jnp.int32),
            pltpu.VMEM((per_w,), jnp.int32),
            pltpu.VMEM((_GK, D // 2), jnp.int32),
            pltpu.VMEM((_GK, D // 2), jnp.int32),
            pltpu.VMEM((_GK, D // 2), jnp.int32),
            pltpu.VMEM((_GK, D // 2), jnp.int32),
            pltpu.SemaphoreType.DMA,
            pltpu.SemaphoreType.DMA,
            pltpu.SemaphoreType.DMA,
            pltpu.SemaphoreType.DMA,
        ],
    )
    return fn(p, q, src, dst)


def _scatter_body(e_len, e_hbm, dst3_hbm, agg_hbm, r0_v, r1_v, idx_v, spmem,
                  sa0, sa1, sd0, sd1):
    cid = lax.axis_index("c")
    sid = lax.axis_index("s")
    per_tile = e_len // 16
    n_iter = per_tile // _SK
    n_sl = _SLAB // _SB

    rows = (r0_v, r1_v)
    asems = (sa0, sa1)
    dsems = (sd0, sd1)

    z16 = jnp.zeros((16,), jnp.float32)

    def zrow(i, carry):
        for j in range(DC // 16):
            r0_v[i, pl.ds(j * 16, 16)] = z16
        return carry

    pltpu.sync_copy(dst3_hbm.at[sid], idx_v)

    for j in range(2):
        chunk = cid * 2 + j

        lax.fori_loop(0, _SK, zrow, 0)
        base_r = sid * _SLAB
        for t in range(_SLAB // _SK):
            pltpu.sync_copy(r0_v, spmem.at[pl.ds(base_r + t * _SK, _SK), :])
        rem = _SLAB % _SK
        if rem:
            pltpu.sync_copy(r0_v.at[pl.ds(0, rem)],
                            spmem.at[pl.ds(base_r + _SLAB - rem, rem), :])
        plsc.subcore_barrier()

        def body(i, carry):
            for b in range(2):
                c = 2 * i + b
                base = sid * per_tile + c * _SK

                @pl.when(i > 0)
                def _():
                    pltpu.make_async_copy(
                        rows[b], spmem.at[pl.ds(0, _SK)], asems[b]).wait()

                pltpu.sync_copy(e_hbm.at[pl.ds(base, _SK), pl.ds(chunk * DC, DC)],
                                rows[b])
                pltpu.async_copy(rows[b], spmem.at[idx_v.at[c]], asems[b],
                                 add=True)
            return carry

        lax.fori_loop(0, n_iter // 2, body, 0)
        for b in range(2):
            pltpu.make_async_copy(rows[b], spmem.at[pl.ds(0, _SK)],
                                  asems[b]).wait()
        plsc.subcore_barrier()

        def drain(t, carry):
            for b in range(2):
                tt = 2 * t + b
                r0 = sid * _SLAB + tt * _SB

                @pl.when(tt < n_sl)
                def _():
                    @pl.when(t > 0)
                    def _():
                        pltpu.make_async_copy(
                            rows[b].at[pl.ds(0, _SB)],
                            agg_hbm.at[pl.ds(0, _SB), pl.ds(0, DC)],
                            dsems[b]).wait()
                    pltpu.sync_copy(spmem.at[pl.ds(r0, _SB), :],
                                    rows[b].at[pl.ds(0, _SB)])
                    pltpu.async_copy(rows[b].at[pl.ds(0, _SB)],
                                     agg_hbm.at[pl.ds(r0, _SB),
                                                pl.ds(chunk * DC, DC)],
                                     dsems[b])
            return carry

        lax.fori_loop(0, (n_sl + 1) // 2, drain, 0)
        for b in range(2):
            pltpu.make_async_copy(rows[b].at[pl.ds(0, _SB)],
                                  agg_hbm.at[pl.ds(0, _SB), pl.ds(0, DC)],
                                  dsems[b]).wait()
        plsc.subcore_barrier()


def _sc_scatter(edges_h, dst3_h, e_len):
    mesh = plsc.VectorSubcoreMesh(core_axis_name="c", subcore_axis_name="s")
    fn = pl.kernel(
        functools.partial(_scatter_body, e_len),
        out_type=jax.ShapeDtypeStruct((N_PAD, D), jnp.float32),
        mesh=mesh,
        scratch_types=[
            pltpu.VMEM((_SK, DC), jnp.float32),
            pltpu.VMEM((_SK, DC), jnp.float32),
            pltpu.VMEM((e_len // 16 // _SK, _SK), jnp.int32),
            pltpu.VMEM_SHARED((N_PAD, DC), jnp.float32),
            pltpu.SemaphoreType.DMA,
            pltpu.SemaphoreType.DMA,
            pltpu.SemaphoreType.DMA,
            pltpu.SemaphoreType.DMA,
        ],
    )
    return fn(edges_h, dst3_h)


def kernel(embedded_mesh_features, embedded_mesh2mesh_edge_features,
           mesh2mesh_edge_indices, embedded_grid2mesh_edge_features,
           grid2mesh_edge_indices, embedded_mesh2grid_edge_features,
           mesh2grid_edge_indices, We1, be1, We2, be2, ln_e_scale, ln_e_bias,
           Wn1, bn1, Wn2, bn2, ln_n_scale, ln_n_bias):
    src = jnp.pad(mesh2mesh_edge_indices[0], (0, E_PAD - N_EDGES))
    dst = jnp.pad(mesh2mesh_edge_indices[1], (0, E_PAD - N_EDGES))
    dst2d = dst.reshape(E_PAD // _SK, _SK)
    nodes = jnp.pad(embedded_mesh_features, ((0, N_PAD - N_NODES), (0, 0)))
    edges = embedded_mesh2mesh_edge_features

    r2 = lambda v: v.reshape(1, D)
    EH = E_PAD // 2
    dst4 = dst.reshape(2, 16, EH // 16 // _SK, _SK)
    dst3_h = (dst4[0], dst4[1])
    e_halves = (edges[:EH], edges[EH:])
    for l in range(NLAYER):
        w1a = We1[l, :D]
        w1b = We1[l, D:2 * D]
        w1c = We1[l, 2 * D:]
        eargs = (w1a, r2(be1[l]), We2[l], r2(be2[l]), r2(ln_e_scale[l]),
                 r2(ln_e_bias[l]))
        p, q = _tc_proj(nodes, w1b, w1c)
        pg0, qg0 = _sc_gather(p, q, src, dst, 0, EH)
        pg1, qg1 = _sc_gather(p, q, src, dst, EH, EH)
        e0 = _tc_edge(e_halves[0], pg0, qg0, *eargs, 0, EH, 0)
        a0 = _sc_scatter(e0, dst3_h[0], EH)
        e1 = _tc_edge(e_halves[1], pg1, qg1, *eargs, EH, EH, 0)
        a1 = _sc_scatter(e1, dst3_h[1], EH)
        e_halves = (e0, e1)
        nodes = _tc_node(nodes, a0, a1, Wn1[l, :D], Wn1[l, D:], r2(bn1[l]),
                         Wn2[l], r2(bn2[l]), r2(ln_n_scale[l]),
                         r2(ln_n_bias[l]))

    nodes_out = nodes[:N_NODES]
    edges_out = jnp.concatenate(e_halves, axis=0)[:N_EDGES]
    return (nodes_out, edges_out, embedded_grid2mesh_edge_features,
            embedded_mesh2grid_edge_features)
